# baseline (speedup 1.0000x reference)
import jax
import jax.numpy as jnp
from jax.experimental import pallas as pl
from jax.experimental.pallas import tpu as pltpu

_BM = 200


def _fused_kernel(adj_ref, x_ref, w0_ref, b0_ref, w12_ref, b12_ref,
                  rec_ref, mu_ref, lv_ref,
                  s_ref, h_ref, zb_ref):
    i = pl.program_id(0)
    p = pl.num_programs(0) // 3

    @pl.when(i == 0)
    def _():
        s_ref[...] = jnp.dot(x_ref[...], w0_ref[...],
                             preferred_element_type=jnp.float32)

    @pl.when(i < p)
    def _():
        acc = jnp.dot(adj_ref[...], s_ref[...],
                      preferred_element_type=jnp.float32)
        h_ref[pl.ds(i * _BM, _BM), :] = jnp.maximum(acc + b0_ref[...], 0.0)

    @pl.when(i == p)
    def _():
        s_ref[...] = jnp.dot(h_ref[...], w12_ref[...],
                             preferred_element_type=jnp.float32)

    @pl.when(jnp.logical_and(i >= p, i < 2 * p))
    def _():
        acc = jnp.dot(adj_ref[...], s_ref[...],
                      preferred_element_type=jnp.float32)
        acc = jnp.maximum(acc + b12_ref[...], 0.0)
        mu = acc[:, :32]
        mu_ref[...] = mu
        lv_ref[...] = acc[:, 32:]
        zb_ref[pl.ds((i - p) * _BM, _BM), :] = mu.astype(jnp.bfloat16)

    @pl.when(i >= 2 * p)
    def _():
        za = zb_ref[pl.ds((i - 2 * p) * _BM, _BM), :]
        prod = jax.lax.dot_general(za, zb_ref[...],
                                   (((1,), (1,)), ((), ())),
                                   preferred_element_type=jnp.float32)
        rec_ref[...] = jax.nn.sigmoid(prod)


def kernel(x, adj, W0, b0, W1, b1, W2, b2):
    n, nfeat = x.shape
    nhid = W0.shape[1]
    zdim = W1.shape[1]
    W12 = jnp.concatenate([W1, W2], axis=1)
    b12 = jnp.concatenate([b1, b2])[None, :]
    p = n // _BM

    def adj_idx(i):
        return (jnp.where(i < p, i, jnp.where(i < 2 * p, i - p, p - 1)), 0)

    def mu_idx(i):
        return (jnp.clip(i - p, 0, p - 1), 0)

    def rec_idx(i):
        return (jnp.clip(i - 2 * p, 0, p - 1), 0)

    adj_recon, mu, logvar = pl.pallas_call(
        _fused_kernel,
        grid=(3 * p,),
        in_specs=[
            pl.BlockSpec((_BM, n), adj_idx),
            pl.BlockSpec((n, nfeat), lambda i: (0, 0)),
            pl.BlockSpec((nfeat, nhid), lambda i: (0, 0)),
            pl.BlockSpec((1, nhid), lambda i: (0, 0)),
            pl.BlockSpec((nhid, 2 * zdim), lambda i: (0, 0)),
            pl.BlockSpec((1, 2 * zdim), lambda i: (0, 0)),
        ],
        out_specs=[
            pl.BlockSpec((_BM, n), rec_idx),
            pl.BlockSpec((_BM, zdim), mu_idx),
            pl.BlockSpec((_BM, zdim), mu_idx),
        ],
        out_shape=[
            jax.ShapeDtypeStruct((n, n), jnp.float32),
            jax.ShapeDtypeStruct((n, zdim), jnp.float32),
            jax.ShapeDtypeStruct((n, zdim), jnp.float32),
        ],
        scratch_shapes=[
            pltpu.VMEM((n, nhid), jnp.float32),
            pltpu.VMEM((n, nhid), jnp.float32),
            pltpu.VMEM((n, zdim), jnp.bfloat16),
        ],
        compiler_params=pltpu.CompilerParams(
            dimension_semantics=("arbitrary",)),
    )(adj, x, W0, b0[None, :], W12, b12)

    return (adj_recon, mu, mu, logvar)


# speedup vs baseline: 1.0124x; 1.0124x over previous
import jax
import jax.numpy as jnp
from jax.experimental import pallas as pl
from jax.experimental.pallas import tpu as pltpu


def _prop1_kernel(adj_ref, x_ref, w_ref, b_ref, o_ref, s_ref):
    @pl.when(pl.program_id(0) == 0)
    def _():
        s_ref[...] = jnp.dot(x_ref[...], w_ref[...],
                             preferred_element_type=jnp.float32)

    acc = jnp.dot(adj_ref[...], s_ref[...],
                  preferred_element_type=jnp.float32)
    o_ref[...] = jnp.maximum(acc + b_ref[...], 0.0)


def _prop2_kernel(adj_ref, h_ref, w_ref, b_ref, mu_ref, lv_ref, s_ref):
    @pl.when(pl.program_id(0) == 0)
    def _():
        s_ref[...] = jnp.dot(h_ref[...], w_ref[...],
                             preferred_element_type=jnp.float32)

    acc = jnp.dot(adj_ref[...], s_ref[...],
                  preferred_element_type=jnp.float32)
    acc = jnp.maximum(acc + b_ref[...], 0.0)
    mu_ref[...] = acc[:, :32]
    lv_ref[...] = acc[:, 32:]


def _dec_kernel(za_ref, zb_ref, o_ref, zb_bf_ref):
    @pl.when(pl.program_id(0) == 0)
    def _():
        zb_bf_ref[...] = zb_ref[...].astype(jnp.bfloat16)

    p = jax.lax.dot_general(za_ref[...].astype(jnp.bfloat16), zb_bf_ref[...],
                            (((1,), (1,)), ((), ())),
                            preferred_element_type=jnp.float32)
    o_ref[...] = jax.nn.sigmoid(p)


_BM = 400


def kernel(x, adj, W0, b0, W1, b1, W2, b2):
    n, nfeat = x.shape
    nhid = W0.shape[1]
    zdim = W1.shape[1]
    W12 = jnp.concatenate([W1, W2], axis=1)
    b12 = jnp.concatenate([b1, b2])[None, :]
    grid = (n // _BM,)
    seq = pltpu.CompilerParams(dimension_semantics=("arbitrary",))

    h = pl.pallas_call(
        _prop1_kernel,
        grid=grid,
        in_specs=[
            pl.BlockSpec((_BM, n), lambda i: (i, 0)),
            pl.BlockSpec((n, nfeat), lambda i: (0, 0)),
            pl.BlockSpec((nfeat, nhid), lambda i: (0, 0)),
            pl.BlockSpec((1, nhid), lambda i: (0, 0)),
        ],
        out_specs=pl.BlockSpec((_BM, nhid), lambda i: (i, 0)),
        out_shape=jax.ShapeDtypeStruct((n, nhid), jnp.float32),
        scratch_shapes=[pltpu.VMEM((n, nhid), jnp.float32)],
        compiler_params=seq,
    )(adj, x, W0, b0[None, :])

    mu, logvar = pl.pallas_call(
        _prop2_kernel,
        grid=grid,
        in_specs=[
            pl.BlockSpec((_BM, n), lambda i: (i, 0)),
            pl.BlockSpec((n, nhid), lambda i: (0, 0)),
            pl.BlockSpec((nhid, 2 * zdim), lambda i: (0, 0)),
            pl.BlockSpec((1, 2 * zdim), lambda i: (0, 0)),
        ],
        out_specs=[
            pl.BlockSpec((_BM, zdim), lambda i: (i, 0)),
            pl.BlockSpec((_BM, zdim), lambda i: (i, 0)),
        ],
        out_shape=[
            jax.ShapeDtypeStruct((n, zdim), jnp.float32),
            jax.ShapeDtypeStruct((n, zdim), jnp.float32),
        ],
        scratch_shapes=[pltpu.VMEM((n, 2 * zdim), jnp.float32)],
        compiler_params=seq,
    )(adj, h, W12, b12)

    adj_recon = pl.pallas_call(
        _dec_kernel,
        grid=grid,
        in_specs=[
            pl.BlockSpec((_BM, zdim), lambda i: (i, 0)),
            pl.BlockSpec((n, zdim), lambda i: (0, 0)),
        ],
        out_specs=pl.BlockSpec((_BM, n), lambda i: (i, 0)),
        out_shape=jax.ShapeDtypeStruct((n, n), jnp.float32),
        scratch_shapes=[pltpu.VMEM((n, zdim), jnp.bfloat16)],
        compiler_params=seq,
    )(mu, mu)

    return (adj_recon, mu, mu, logvar)
